# baseline (speedup 1.0000x reference)
import functools

import jax
import jax.numpy as jnp
from jax import lax
from jax.experimental import pallas as pl
from jax.experimental.pallas import tpu as pltpu
from jax.experimental.pallas import tpu_sc as plsc

_A = 128
_DM = 4096
_RK = 64
_B = 64
_NW = 32
_IDW = _B // _NW
_BANDS = _RK // 8
_CHALF = _DM // 2


def _body(w_hbm, idx_hbm, out_hbm,
          idx_v, buf0, buf1, buf2,
          sem_g0, sem_g1, sem_g2, sem_w0, sem_w1, sem_w2):
    wid = lax.axis_index("s") * 2 + lax.axis_index("c")

    pltpu.sync_copy(idx_hbm.at[wid], idx_v)

    def src(c):
        i, band = c % _IDW, c // _IDW
        return w_hbm.at[idx_v.at[i], pl.ds(band * 8, 8), :]

    def dst(c):
        i, band = c % _IDW, c // _IDW
        return out_hbm.at[pl.ds(wid * _IDW + i, 1), pl.ds(band * 8, 8), :]

    nchunks = _BANDS * 2
    bufs = (buf0, buf1, buf2)
    gsems = (sem_g0, sem_g1, sem_g2)
    wsems = (sem_w0, sem_w1, sem_w2)
    nbuf = 3
    gathers = [None] * nbuf
    writes = [None] * nbuf

    gathers[0] = pltpu.async_copy(src(0), bufs[0], gsems[0])
    gathers[1] = pltpu.async_copy(src(1), bufs[1], gsems[1])
    for c in range(nchunks):
        s = c % nbuf
        nxt = c + 2
        if nxt < nchunks:
            sn = nxt % nbuf
            if writes[sn] is not None:
                writes[sn].wait()
            gathers[sn] = pltpu.async_copy(src(nxt), bufs[sn], gsems[sn])
        gathers[s].wait()
        writes[s] = pltpu.async_copy(bufs[s], dst(c), wsems[s])
    for s in range(nbuf):
        writes[s].wait()


@jax.jit
def _sc_gather(wv, idx2):
    mesh = plsc.VectorSubcoreMesh(core_axis_name="c", subcore_axis_name="s")
    f = functools.partial(
        pl.kernel,
        mesh=mesh,
        out_type=jax.ShapeDtypeStruct((_B, _RK, _DM), jnp.float32),
        scratch_types=[
            pltpu.VMEM((_IDW, 1), jnp.int32),
            pltpu.VMEM((1, 8, _DM), jnp.float32),
            pltpu.VMEM((1, 8, _DM), jnp.float32),
            pltpu.VMEM((1, 8, _DM), jnp.float32),
            pltpu.SemaphoreType.DMA,
            pltpu.SemaphoreType.DMA,
            pltpu.SemaphoreType.DMA,
            pltpu.SemaphoreType.DMA,
            pltpu.SemaphoreType.DMA,
            pltpu.SemaphoreType.DMA,
        ],
    )(_body)
    return f(wv, idx2)


def kernel(weight, adapter_ids):
    wv = jnp.swapaxes(weight, 1, 2)
    idx2 = adapter_ids.astype(jnp.int32).reshape(_NW, _IDW, 1)
    out = _sc_gather(wv, idx2)
    return jnp.swapaxes(out, 1, 2)
